# Initial kernel scaffold; baseline (speedup 1.0000x reference)
import jax
import jax.numpy as jnp
from jax.experimental import pallas as pl


def kernel(x, centers, intercept):
    raise NotImplementedError("write your pallas kernel here")


# speedup vs baseline: 6.2527x; 6.2527x over previous
import functools

import jax
import jax.numpy as jnp
from jax.experimental import pallas as pl

IN_DIM_K = 96
NB_K = 512
SPARS = 32
TILE_P = 1024


def _body(x_ref, c_ref, b_ref, gsum_ref, label_ref, *, n_p_tiles, n_pixels):
    p = pl.program_id(1)

    xblk = x_ref[0]
    logits = jax.lax.dot_general(
        xblk, c_ref[...], (((0,), (0,)), ((), ())),
        preferred_element_type=jnp.float32)
    sim = jax.nn.sigmoid(logits + b_ref[0, 0])

    m = jnp.max(sim, axis=-1, keepdims=True)
    iota_k = jax.lax.broadcasted_iota(jnp.int32, sim.shape, 1)
    label = jnp.min(jnp.where(sim == m, iota_k, NB_K), axis=-1)
    label_ref[0, 0, 0, :] = label.astype(jnp.int32)

    v = jax.lax.bitcast_convert_type(sim, jnp.int32)
    prefix = jnp.zeros((sim.shape[0], 1), jnp.int32)
    for bit in range(29, -1, -1):
        cand = prefix | (1 << bit)
        cnt = jnp.sum((v >= cand).astype(jnp.int32), axis=-1, keepdims=True)
        prefix = jnp.where(cnt >= SPARS, cand, prefix)

    masked = jnp.where(v >= prefix, sim, 0.0)
    contrib = jnp.sum(masked, axis=0)

    @pl.when(p == 0)
    def _():
        gsum_ref[0, 0, :] = contrib

    @pl.when(p != 0)
    def _():
        gsum_ref[0, 0, :] = gsum_ref[0, 0, :] + contrib

    @pl.when(p == n_p_tiles - 1)
    def _():
        gsum_ref[0, 0, :] = gsum_ref[0, 0, :] * (1.0 / n_pixels)


def kernel(x, centers, intercept):
    B, C, H, W = x.shape
    P = H * W
    n_p_tiles = P // TILE_P
    xr = x.reshape(B, C, P)

    gsum, label = pl.pallas_call(
        functools.partial(_body, n_p_tiles=n_p_tiles, n_pixels=P),
        grid=(B, n_p_tiles),
        in_specs=[
            pl.BlockSpec((1, C, TILE_P), lambda b, p: (b, 0, p)),
            pl.BlockSpec((C, NB_K), lambda b, p: (0, 0)),
            pl.BlockSpec((1, 1), lambda b, p: (0, 0)),
        ],
        out_specs=[
            pl.BlockSpec((1, 1, NB_K), lambda b, p: (b, 0, 0)),
            pl.BlockSpec((1, 1, 1, TILE_P), lambda b, p: (b, p, 0, 0)),
        ],
        out_shape=[
            jax.ShapeDtypeStruct((B, 1, NB_K), jnp.float32),
            jax.ShapeDtypeStruct((B, n_p_tiles, 1, TILE_P), jnp.int32),
        ],
    )(xr, centers, intercept.reshape(1, 1))

    return gsum.reshape(B, NB_K), label.reshape(B, H, W)


# speedup vs baseline: 9.0252x; 1.4434x over previous
import functools

import jax
import jax.numpy as jnp
from jax.experimental import pallas as pl

IN_DIM_K = 96
NB_K = 512
SPARS = 32
TILE_P = 2048


def _body(x_ref, c_ref, b_ref, gsum_ref, label_ref, *, n_p_tiles, n_pixels):
    p = pl.program_id(1)

    xblk = x_ref[0]
    logits = jax.lax.dot_general(
        xblk, c_ref[...], (((0,), (0,)), ((), ())),
        preferred_element_type=jnp.float32)
    sim = jax.nn.sigmoid(logits + b_ref[0, 0])

    m = jnp.max(sim, axis=-1, keepdims=True)
    iota_k = jax.lax.broadcasted_iota(jnp.int32, sim.shape, 1)
    label = jnp.min(jnp.where(sim == m, iota_k, NB_K), axis=-1)
    label_ref[0, 0, 0, :] = label.astype(jnp.int32)

    v = jax.lax.bitcast_convert_type(sim, jnp.int32)
    prefix = jnp.zeros((sim.shape[0], 1), jnp.int32)
    for bit in range(29, -1, -1):
        cand = prefix | (1 << bit)
        cnt = jnp.sum((v >= cand).astype(jnp.float32), axis=-1, keepdims=True)
        prefix = jnp.where(cnt >= float(SPARS), cand, prefix)

    masked = jnp.where(v >= prefix, sim, 0.0)
    contrib = jnp.sum(masked, axis=0)

    @pl.when(p == 0)
    def _():
        gsum_ref[0, 0, :] = contrib

    @pl.when(p != 0)
    def _():
        gsum_ref[0, 0, :] = gsum_ref[0, 0, :] + contrib

    @pl.when(p == n_p_tiles - 1)
    def _():
        gsum_ref[0, 0, :] = gsum_ref[0, 0, :] * (1.0 / n_pixels)


def kernel(x, centers, intercept):
    B, C, H, W = x.shape
    P = H * W
    n_p_tiles = P // TILE_P
    xr = x.reshape(B, C, P)

    gsum, label = pl.pallas_call(
        functools.partial(_body, n_p_tiles=n_p_tiles, n_pixels=P),
        grid=(B, n_p_tiles),
        in_specs=[
            pl.BlockSpec((1, C, TILE_P), lambda b, p: (b, 0, p)),
            pl.BlockSpec((C, NB_K), lambda b, p: (0, 0)),
            pl.BlockSpec((1, 1), lambda b, p: (0, 0)),
        ],
        out_specs=[
            pl.BlockSpec((1, 1, NB_K), lambda b, p: (b, 0, 0)),
            pl.BlockSpec((1, 1, 1, TILE_P), lambda b, p: (b, p, 0, 0)),
        ],
        out_shape=[
            jax.ShapeDtypeStruct((B, 1, NB_K), jnp.float32),
            jax.ShapeDtypeStruct((B, n_p_tiles, 1, TILE_P), jnp.int32),
        ],
    )(xr, centers, intercept.reshape(1, 1))

    return gsum.reshape(B, NB_K), label.reshape(B, H, W)
